# baseline (speedup 1.0000x reference)
import functools

import jax
import jax.numpy as jnp
from jax import lax
from jax.experimental import pallas as pl
from jax.experimental.pallas import tpu as pltpu
from jax.experimental.pallas import tpu_sc as plsc

BATCH = 16384
F = 4096
L = 16
NW = 32
ROWS_PER_TILE = BATCH // NW
CHUNK = 8
NCHUNKS = ROWS_PER_TILE // CHUNK
JBLOCKS = F // L


def _permute_body(in_hbm, perm_hbm, out_hbm, perm_v, in_v, out_v):
    wid = lax.axis_index("s") * 2 + lax.axis_index("c")
    base = wid * ROWS_PER_TILE
    pltpu.sync_copy(perm_hbm, perm_v)

    def chunk_body(c, _):
        r0 = base + c * CHUNK
        pltpu.sync_copy(in_hbm.at[pl.ds(r0, CHUNK)], in_v)

        def j_body(j, _):
            col0 = pl.multiple_of(j * L, L)
            idx = perm_v[pl.ds(col0, L)]
            for r in range(CHUNK):
                row_idx = jnp.full((L,), r, dtype=jnp.int32)
                vals = plsc.load_gather(in_v, [row_idx, idx])
                out_v[r, pl.ds(col0, L)] = vals
            return 0

        lax.fori_loop(0, JBLOCKS, j_body, 0)
        pltpu.sync_copy(out_v, out_hbm.at[pl.ds(r0, CHUNK)])
        return 0

    lax.fori_loop(0, NCHUNKS, chunk_body, 0)


@functools.partial(
    pl.kernel,
    mesh=plsc.VectorSubcoreMesh(core_axis_name="c", subcore_axis_name="s"),
    out_type=jax.ShapeDtypeStruct((BATCH, F), jnp.float32),
    scratch_types=[
        pltpu.VMEM((F,), jnp.int32),
        pltpu.VMEM((CHUNK, F), jnp.float32),
        pltpu.VMEM((CHUNK, F), jnp.float32),
    ],
    compiler_params=pltpu.CompilerParams(
        needs_layout_passes=False, use_tc_tiling_on_sc=False
    ),
)
def _permute_kernel(in_hbm, perm_hbm, out_hbm, perm_v, in_v, out_v):
    _permute_body(in_hbm, perm_hbm, out_hbm, perm_v, in_v, out_v)


def kernel(inputs, permutation):
    outputs = _permute_kernel(inputs, permutation.astype(jnp.int32))
    logabsdet = jnp.zeros((inputs.shape[0],), dtype=inputs.dtype)
    return (outputs, logabsdet)


# speedup vs baseline: 3.5323x; 3.5323x over previous
import functools

import jax
import jax.numpy as jnp
from jax import lax
from jax.experimental import pallas as pl
from jax.experimental.pallas import tpu as pltpu
from jax.experimental.pallas import tpu_sc as plsc

BATCH = 16384
F = 4096
L = 16
NW = 32
ROWS_PER_TILE = BATCH // NW
CHUNK = 8
NCHUNKS = ROWS_PER_TILE // CHUNK
JBLOCKS = F // L


def _permute_body(in_hbm, perm_hbm, out_hbm, perm_v, in_v, out_v):
    wid = lax.axis_index("s") * 2 + lax.axis_index("c")
    base = wid * ROWS_PER_TILE
    pltpu.sync_copy(perm_hbm, perm_v)

    def chunk_body(c, _):
        r0 = base + c * CHUNK
        pltpu.sync_copy(in_hbm.at[pl.ds(r0, CHUNK)], in_v)

        @plsc.parallel_loop(0, JBLOCKS, unroll=4)
        def j_body(j):
            col0 = pl.multiple_of(j * L, L)
            idx = perm_v[pl.ds(col0, L)]
            for r in range(CHUNK):
                row_idx = jnp.full((L,), r, dtype=jnp.int32)
                vals = plsc.load_gather(in_v, [row_idx, idx])
                out_v[r, pl.ds(col0, L)] = vals

        pltpu.sync_copy(out_v, out_hbm.at[pl.ds(r0, CHUNK)])
        return 0

    lax.fori_loop(0, NCHUNKS, chunk_body, 0)


@functools.partial(
    pl.kernel,
    mesh=plsc.VectorSubcoreMesh(core_axis_name="c", subcore_axis_name="s"),
    out_type=jax.ShapeDtypeStruct((BATCH, F), jnp.float32),
    scratch_types=[
        pltpu.VMEM((F,), jnp.int32),
        pltpu.VMEM((CHUNK, F), jnp.float32),
        pltpu.VMEM((CHUNK, F), jnp.float32),
    ],
    compiler_params=pltpu.CompilerParams(needs_layout_passes=False),
)
def _permute_kernel(in_hbm, perm_hbm, out_hbm, perm_v, in_v, out_v):
    _permute_body(in_hbm, perm_hbm, out_hbm, perm_v, in_v, out_v)


def kernel(inputs, permutation):
    outputs = _permute_kernel(inputs, permutation.astype(jnp.int32))
    logabsdet = jnp.zeros((inputs.shape[0],), dtype=inputs.dtype)
    return (outputs, logabsdet)


# speedup vs baseline: 5.8320x; 1.6510x over previous
import functools

import jax
import jax.numpy as jnp
from jax import lax
from jax.experimental import pallas as pl
from jax.experimental.pallas import tpu as pltpu
from jax.experimental.pallas import tpu_sc as plsc

BATCH = 16384
F = 4096
L = 16
NW = 32
ROWS_PER_TILE = BATCH // NW
CHUNK = 4
NCHUNKS = ROWS_PER_TILE // CHUNK
NG = NCHUNKS // 2
JBLOCKS = F // L


def _gather_chunk(perm_v, in_v, out_v):
    @plsc.parallel_loop(0, JBLOCKS, unroll=4)
    def j_body(j):
        col0 = pl.multiple_of(j * L, L)
        idx = perm_v[pl.ds(col0, L)]
        for r in range(CHUNK):
            row_idx = jnp.full((L,), r, dtype=jnp.int32)
            vals = plsc.load_gather(in_v, [row_idx, idx])
            out_v[r, pl.ds(col0, L)] = vals


def _permute_body(in_hbm, perm_hbm, out_hbm, perm_v, in_bufs, out_bufs, sems):
    wid = lax.axis_index("s") * 2 + lax.axis_index("c")
    base = wid * ROWS_PER_TILE
    in_sems, out_sems = sems[:2], sems[2:]
    pltpu.sync_copy(perm_hbm, perm_v)

    def rows(c):
        return pl.ds(base + c * CHUNK, CHUNK)

    def start_in(c, b):
        pltpu.async_copy(in_hbm.at[rows(c)], in_bufs[b], in_sems[b])

    def wait_in(c, b):
        pltpu.make_async_copy(in_hbm.at[rows(c)], in_bufs[b], in_sems[b]).wait()

    def start_out(c, b):
        pltpu.async_copy(out_bufs[b], out_hbm.at[rows(c)], out_sems[b])

    def wait_out(c, b):
        pltpu.make_async_copy(out_bufs[b], out_hbm.at[rows(c)], out_sems[b]).wait()

    start_in(0, 0)
    start_in(1, 1)
    for b in (0, 1):
        wait_in(b, b)
        _gather_chunk(perm_v, in_bufs[b], out_bufs[b])
        start_out(b, b)
        start_in(b + 2, b)

    def group_body(g, _):
        for b in (0, 1):
            c = 2 * g + b
            wait_in(c, b)
            wait_out(c - 2, b)
            _gather_chunk(perm_v, in_bufs[b], out_bufs[b])
            start_out(c, b)
            c2 = jnp.minimum(c + 2, NCHUNKS - 1)
            start_in(c2, b)
        return 0

    lax.fori_loop(1, NG, group_body, 0)

    for b in (0, 1):
        wait_in(NCHUNKS - 1, b)
        wait_out(NCHUNKS - 2 + b, b)


@functools.partial(
    pl.kernel,
    mesh=plsc.VectorSubcoreMesh(core_axis_name="c", subcore_axis_name="s"),
    out_type=jax.ShapeDtypeStruct((BATCH, F), jnp.float32),
    scratch_types=[
        pltpu.VMEM((F,), jnp.int32),
        pltpu.VMEM((CHUNK, F), jnp.float32),
        pltpu.VMEM((CHUNK, F), jnp.float32),
        pltpu.VMEM((CHUNK, F), jnp.float32),
        pltpu.VMEM((CHUNK, F), jnp.float32),
        pltpu.SemaphoreType.DMA,
        pltpu.SemaphoreType.DMA,
        pltpu.SemaphoreType.DMA,
        pltpu.SemaphoreType.DMA,
    ],
    compiler_params=pltpu.CompilerParams(needs_layout_passes=False),
)
def _permute_kernel(in_hbm, perm_hbm, out_hbm, perm_v,
                    in0, in1, out0, out1, s0, s1, s2, s3):
    _permute_body(in_hbm, perm_hbm, out_hbm, perm_v,
                  (in0, in1), (out0, out1), (s0, s1, s2, s3))


def kernel(inputs, permutation):
    outputs = _permute_kernel(inputs, permutation.astype(jnp.int32))
    logabsdet = jnp.zeros((inputs.shape[0],), dtype=inputs.dtype)
    return (outputs, logabsdet)


# speedup vs baseline: 5.8721x; 1.0069x over previous
import functools

import jax
import jax.numpy as jnp
from jax import lax
from jax.experimental import pallas as pl
from jax.experimental.pallas import tpu as pltpu
from jax.experimental.pallas import tpu_sc as plsc

BATCH = 16384
F = 4096
L = 16
NW = 32
ROWS_PER_TILE = BATCH // NW
CHUNK = 4
NCHUNKS = ROWS_PER_TILE // CHUNK
NG = NCHUNKS // 2
JBLOCKS = F // L


def _gather_chunk(perm_v, in_v, out_v):
    @plsc.parallel_loop(0, JBLOCKS, unroll=8)
    def j_body(j):
        col0 = pl.multiple_of(j * L, L)
        idx = perm_v[pl.ds(col0, L)]
        for r in range(CHUNK):
            row_idx = jnp.full((L,), r, dtype=jnp.int32)
            vals = plsc.load_gather(in_v, [row_idx, idx])
            out_v[r, pl.ds(col0, L)] = vals


def _permute_body(in_hbm, perm_hbm, out_hbm, perm_v, in_bufs, out_bufs, sems):
    wid = lax.axis_index("s") * 2 + lax.axis_index("c")
    base = wid * ROWS_PER_TILE
    in_sems, out_sems = sems[:2], sems[2:]
    pltpu.sync_copy(perm_hbm, perm_v)

    def rows(c):
        return pl.ds(base + c * CHUNK, CHUNK)

    def start_in(c, b):
        pltpu.async_copy(in_hbm.at[rows(c)], in_bufs[b], in_sems[b])

    def wait_in(c, b):
        pltpu.make_async_copy(in_hbm.at[rows(c)], in_bufs[b], in_sems[b]).wait()

    def start_out(c, b):
        pltpu.async_copy(out_bufs[b], out_hbm.at[rows(c)], out_sems[b])

    def wait_out(c, b):
        pltpu.make_async_copy(out_bufs[b], out_hbm.at[rows(c)], out_sems[b]).wait()

    start_in(0, 0)
    start_in(1, 1)
    for b in (0, 1):
        wait_in(b, b)
        _gather_chunk(perm_v, in_bufs[b], out_bufs[b])
        start_out(b, b)
        start_in(b + 2, b)

    def group_body(g, _):
        for b in (0, 1):
            c = 2 * g + b
            wait_in(c, b)
            wait_out(c - 2, b)
            _gather_chunk(perm_v, in_bufs[b], out_bufs[b])
            start_out(c, b)
            c2 = jnp.minimum(c + 2, NCHUNKS - 1)
            start_in(c2, b)
        return 0

    lax.fori_loop(1, NG, group_body, 0)

    for b in (0, 1):
        wait_in(NCHUNKS - 1, b)
        wait_out(NCHUNKS - 2 + b, b)


@functools.partial(
    pl.kernel,
    mesh=plsc.VectorSubcoreMesh(core_axis_name="c", subcore_axis_name="s"),
    out_type=jax.ShapeDtypeStruct((BATCH, F), jnp.float32),
    scratch_types=[
        pltpu.VMEM((F,), jnp.int32),
        pltpu.VMEM((CHUNK, F), jnp.float32),
        pltpu.VMEM((CHUNK, F), jnp.float32),
        pltpu.VMEM((CHUNK, F), jnp.float32),
        pltpu.VMEM((CHUNK, F), jnp.float32),
        pltpu.SemaphoreType.DMA,
        pltpu.SemaphoreType.DMA,
        pltpu.SemaphoreType.DMA,
        pltpu.SemaphoreType.DMA,
    ],
    compiler_params=pltpu.CompilerParams(needs_layout_passes=False),
)
def _permute_kernel(in_hbm, perm_hbm, out_hbm, perm_v,
                    in0, in1, out0, out1, s0, s1, s2, s3):
    _permute_body(in_hbm, perm_hbm, out_hbm, perm_v,
                  (in0, in1), (out0, out1), (s0, s1, s2, s3))


def kernel(inputs, permutation):
    outputs = _permute_kernel(inputs, permutation.astype(jnp.int32))
    logabsdet = jnp.zeros((inputs.shape[0],), dtype=inputs.dtype)
    return (outputs, logabsdet)


# speedup vs baseline: 6.0377x; 1.0282x over previous
import functools

import jax
import jax.numpy as jnp
from jax import lax
from jax.experimental import pallas as pl
from jax.experimental.pallas import tpu as pltpu
from jax.experimental.pallas import tpu_sc as plsc

BATCH = 16384
F = 4096
L = 16
NW = 32
ROWS_PER_TILE = BATCH // NW
CHUNK = 2
NBUF = 4
NCHUNKS = ROWS_PER_TILE // CHUNK
NG = NCHUNKS // NBUF
JBLOCKS = F // L


def _gather_chunk(perm_v, in_v, out_v):
    @plsc.parallel_loop(0, JBLOCKS, unroll=8)
    def j_body(j):
        col0 = pl.multiple_of(j * L, L)
        idx = perm_v[pl.ds(col0, L)]
        for r in range(CHUNK):
            row_idx = jnp.full((L,), r, dtype=jnp.int32)
            vals = plsc.load_gather(in_v, [row_idx, idx])
            out_v[r, pl.ds(col0, L)] = vals


def _permute_body(in_hbm, perm_hbm, out_hbm, perm_v, in_bufs, out_bufs,
                  in_sems, out_sems):
    wid = lax.axis_index("s") * 2 + lax.axis_index("c")
    base = wid * ROWS_PER_TILE
    pltpu.sync_copy(perm_hbm, perm_v)

    def rows(c):
        return pl.ds(base + c * CHUNK, CHUNK)

    def start_in(c, b):
        pltpu.async_copy(in_hbm.at[rows(c)], in_bufs[b], in_sems[b])

    def wait_in(c, b):
        pltpu.make_async_copy(in_hbm.at[rows(c)], in_bufs[b], in_sems[b]).wait()

    def start_out(c, b):
        pltpu.async_copy(out_bufs[b], out_hbm.at[rows(c)], out_sems[b])

    def wait_out(c, b):
        pltpu.make_async_copy(out_bufs[b], out_hbm.at[rows(c)], out_sems[b]).wait()

    for b in range(NBUF):
        start_in(b, b)
    for b in range(NBUF):
        wait_in(b, b)
        _gather_chunk(perm_v, in_bufs[b], out_bufs[b])
        start_out(b, b)
        start_in(b + NBUF, b)

    def group_body(g, _):
        for b in range(NBUF):
            c = NBUF * g + b
            wait_in(c, b)
            wait_out(c - NBUF, b)
            _gather_chunk(perm_v, in_bufs[b], out_bufs[b])
            start_out(c, b)
            c2 = jnp.minimum(c + NBUF, NCHUNKS - 1)
            start_in(c2, b)
        return 0

    lax.fori_loop(1, NG, group_body, 0)

    for b in range(NBUF):
        wait_in(NCHUNKS - 1, b)
        wait_out(NCHUNKS - NBUF + b, b)


@functools.partial(
    pl.kernel,
    mesh=plsc.VectorSubcoreMesh(core_axis_name="c", subcore_axis_name="s"),
    out_type=jax.ShapeDtypeStruct((BATCH, F), jnp.float32),
    scratch_types=(
        [pltpu.VMEM((F,), jnp.int32)]
        + [pltpu.VMEM((CHUNK, F), jnp.float32)] * (2 * NBUF)
        + [pltpu.SemaphoreType.DMA] * (2 * NBUF)
    ),
    compiler_params=pltpu.CompilerParams(needs_layout_passes=False),
)
def _permute_kernel(in_hbm, perm_hbm, out_hbm, perm_v, *rest):
    bufs, sems = rest[: 2 * NBUF], rest[2 * NBUF:]
    _permute_body(
        in_hbm, perm_hbm, out_hbm, perm_v,
        bufs[:NBUF], bufs[NBUF:], sems[:NBUF], sems[NBUF:],
    )


def kernel(inputs, permutation):
    outputs = _permute_kernel(inputs, permutation.astype(jnp.int32))
    logabsdet = jnp.zeros((inputs.shape[0],), dtype=inputs.dtype)
    return (outputs, logabsdet)
